# Initial kernel scaffold; baseline (speedup 1.0000x reference)
import jax
import jax.numpy as jnp
from jax.experimental import pallas as pl


def kernel(center, context, neg_context, in_embed_w, out_embed_w):
    raise NotImplementedError("write your pallas kernel here")


# speedup vs baseline: 1.5882x; 1.5882x over previous
import functools

import jax
import jax.numpy as jnp
from jax import lax
from jax.experimental import pallas as pl
from jax.experimental.pallas import tpu as pltpu
from jax.experimental.pallas import tpu_sc as plsc

VOCAB_ = 1000000
DIM = 64
BATCH = 16384
KNEG = 5

NUM_CORES = 2
NUM_SUBCORES = 16
NW = NUM_CORES * NUM_SUBCORES

CH = BATCH // NW
NEG_PER_W = BATCH * KNEG // NW
NEG_CHUNKS = NEG_PER_W // CH


@jax.jit
def _sc_gather_all(in_w, out_w, center, context, negf):
    mesh = plsc.VectorSubcoreMesh(core_axis_name="c", subcore_axis_name="s")

    @functools.partial(
        pl.kernel,
        mesh=mesh,
        out_type=(
            jax.ShapeDtypeStruct((BATCH, DIM), jnp.float32),
            jax.ShapeDtypeStruct((BATCH, DIM), jnp.float32),
            jax.ShapeDtypeStruct((BATCH * KNEG, DIM), jnp.float32),
        ),
        scratch_types=[
            pltpu.VMEM((CH,), jnp.int32),
            pltpu.VMEM((CH, DIM), jnp.float32),
            pltpu.SemaphoreType.DMA,
        ],
        compiler_params=pltpu.CompilerParams(use_tc_tiling_on_sc=False),
    )
    def k(in_hbm, out_hbm, c_hbm, x_hbm, n_hbm, vc_hbm, ctx_hbm, neg_hbm,
          idx_v, rows_v, sem):
        wid = lax.axis_index("s") * NUM_CORES + lax.axis_index("c")
        base = wid * CH
        pltpu.sync_copy(c_hbm.at[pl.ds(base, CH)], idx_v)
        pltpu.async_copy(in_hbm.at[idx_v], rows_v, sem).wait()
        pltpu.sync_copy(rows_v, vc_hbm.at[pl.ds(base, CH)])
        pltpu.sync_copy(x_hbm.at[pl.ds(base, CH)], idx_v)
        pltpu.async_copy(out_hbm.at[idx_v], rows_v, sem).wait()
        pltpu.sync_copy(rows_v, ctx_hbm.at[pl.ds(base, CH)])
        nbase = wid * NEG_PER_W

        @pl.loop(0, NEG_CHUNKS)
        def _(j):
            off = nbase + j * CH
            pltpu.sync_copy(n_hbm.at[pl.ds(off, CH)], idx_v)
            pltpu.async_copy(out_hbm.at[idx_v], rows_v, sem).wait()
            pltpu.sync_copy(rows_v, neg_hbm.at[pl.ds(off, CH)])

    return k(in_w, out_w, center, context, negf)


def _softplus(x):
    return jnp.maximum(x, 0.0) + jnp.log1p(jnp.exp(-jnp.abs(x)))


def _score_body(vc_ref, ctx_ref, neg_ref, o_ref):
    i = pl.program_id(0)
    v = vc_ref[...]
    c = ctx_ref[...]
    n = neg_ref[...]
    pos = jnp.sum(v * c, axis=1)
    pos_l = jnp.sum(_softplus(-pos))
    ns = jnp.sum(n * v[:, None, :], axis=-1)
    neg_l = jnp.sum(_softplus(ns))

    @pl.when(i == 0)
    def _():
        o_ref[...] = jnp.zeros((1, 1), jnp.float32)

    o_ref[...] += jnp.full((1, 1), pos_l + neg_l, jnp.float32)


@jax.jit
def _tc_score(vc, ctx, neg):
    Bb = 2048
    grid = (BATCH // Bb,)
    out = pl.pallas_call(
        _score_body,
        grid=grid,
        in_specs=[
            pl.BlockSpec((Bb, DIM), lambda i: (i, 0)),
            pl.BlockSpec((Bb, DIM), lambda i: (i, 0)),
            pl.BlockSpec((Bb, KNEG, DIM), lambda i: (i, 0, 0)),
        ],
        out_specs=pl.BlockSpec((1, 1), lambda i: (0, 0)),
        out_shape=jax.ShapeDtypeStruct((1, 1), jnp.float32),
    )(vc, ctx, neg)
    return out[0, 0] / BATCH


def kernel(center, context, neg_context, in_embed_w, out_embed_w):
    center = center.astype(jnp.int32)
    context = context.astype(jnp.int32)
    negf = neg_context.reshape(-1).astype(jnp.int32)
    vc, ctx, neg = _sc_gather_all(in_embed_w, out_embed_w, center, context,
                                  negf)
    return _tc_score(vc, ctx, neg.reshape(BATCH, KNEG, DIM))


# speedup vs baseline: 1.5919x; 1.0023x over previous
import functools

import jax
import jax.numpy as jnp
from jax import lax
from jax.experimental import pallas as pl
from jax.experimental.pallas import tpu as pltpu
from jax.experimental.pallas import tpu_sc as plsc

VOCAB_ = 1000000
DIM = 64
BATCH = 16384
KNEG = 5

NUM_CORES = 2
NUM_SUBCORES = 16
NW = NUM_CORES * NUM_SUBCORES

CH = BATCH // NW
NEG_PER_W = BATCH * KNEG // NW
NEG_CHUNKS = NEG_PER_W // CH
NCHUNK = 2 + NEG_CHUNKS
NBUF = 3
IDX_PER_W = CH * NCHUNK


@jax.jit
def _sc_gather_all(in_w, out_w, center, context, negf):
    mesh = plsc.VectorSubcoreMesh(core_axis_name="c", subcore_axis_name="s")

    @functools.partial(
        pl.kernel,
        mesh=mesh,
        out_type=(
            jax.ShapeDtypeStruct((BATCH, DIM), jnp.float32),
            jax.ShapeDtypeStruct((BATCH, DIM), jnp.float32),
            jax.ShapeDtypeStruct((BATCH * KNEG, DIM), jnp.float32),
        ),
        scratch_types=[
            pltpu.VMEM((IDX_PER_W,), jnp.int32),
            pltpu.VMEM((NBUF, CH, DIM), jnp.float32),
            pltpu.SemaphoreType.DMA((NBUF,)),
            pltpu.SemaphoreType.DMA((NBUF,)),
        ],
        compiler_params=pltpu.CompilerParams(use_tc_tiling_on_sc=False),
    )
    def k(in_hbm, out_hbm, c_hbm, x_hbm, n_hbm, vc_hbm, ctx_hbm, neg_hbm,
          idx_v, bufs, gsems, wsems):
        wid = lax.axis_index("s") * NUM_CORES + lax.axis_index("c")
        base = wid * CH
        nbase = wid * NEG_PER_W
        pltpu.sync_copy(c_hbm.at[pl.ds(base, CH)], idx_v.at[pl.ds(0, CH)])
        pltpu.sync_copy(x_hbm.at[pl.ds(base, CH)], idx_v.at[pl.ds(CH, CH)])
        pltpu.sync_copy(n_hbm.at[pl.ds(nbase, NEG_PER_W)],
                        idx_v.at[pl.ds(2 * CH, NEG_PER_W)])
        chunks = [(in_hbm, 0, vc_hbm, base), (out_hbm, CH, ctx_hbm, base)]
        for j in range(NEG_CHUNKS):
            chunks.append((out_hbm, (2 + j) * CH, neg_hbm, nbase + j * CH))
        gcopies = [None] * NCHUNK
        wcopies = [None] * NCHUNK
        for c, (tbl, ioff, dst, doff) in enumerate(chunks):
            s = c % NBUF
            if c >= NBUF:
                wcopies[c - NBUF].wait()
            gcopies[c] = pltpu.async_copy(
                tbl.at[idx_v.at[pl.ds(ioff, CH)]], bufs.at[s], gsems.at[s])
            if c > 0:
                p = c - 1
                gcopies[p].wait()
                wcopies[p] = pltpu.async_copy(
                    bufs.at[p % NBUF],
                    chunks[p][2].at[pl.ds(chunks[p][3], CH)],
                    wsems.at[p % NBUF])
        last = NCHUNK - 1
        gcopies[last].wait()
        wcopies[last] = pltpu.async_copy(
            bufs.at[last % NBUF],
            chunks[last][2].at[pl.ds(chunks[last][3], CH)],
            wsems.at[last % NBUF])
        for c in range(max(0, NCHUNK - NBUF), NCHUNK):
            wcopies[c].wait()

    return k(in_w, out_w, center, context, negf)


def _softplus(x):
    return jnp.maximum(x, 0.0) + jnp.log1p(jnp.exp(-jnp.abs(x)))


def _score_body(vc_ref, ctx_ref, neg_ref, o_ref):
    i = pl.program_id(0)
    v = vc_ref[...]
    c = ctx_ref[...]
    n = neg_ref[...]
    pos = jnp.sum(v * c, axis=1)
    pos_l = jnp.sum(_softplus(-pos))
    ns = jnp.sum(n * v[:, None, :], axis=-1)
    neg_l = jnp.sum(_softplus(ns))

    @pl.when(i == 0)
    def _():
        o_ref[...] = jnp.zeros((1, 1), jnp.float32)

    o_ref[...] += jnp.full((1, 1), pos_l + neg_l, jnp.float32)


@jax.jit
def _tc_score(vc, ctx, neg):
    Bb = 2048
    grid = (BATCH // Bb,)
    out = pl.pallas_call(
        _score_body,
        grid=grid,
        in_specs=[
            pl.BlockSpec((Bb, DIM), lambda i: (i, 0)),
            pl.BlockSpec((Bb, DIM), lambda i: (i, 0)),
            pl.BlockSpec((Bb, KNEG, DIM), lambda i: (i, 0, 0)),
        ],
        out_specs=pl.BlockSpec((1, 1), lambda i: (0, 0)),
        out_shape=jax.ShapeDtypeStruct((1, 1), jnp.float32),
    )(vc, ctx, neg)
    return out[0, 0] / BATCH


def kernel(center, context, neg_context, in_embed_w, out_embed_w):
    center = center.astype(jnp.int32)
    context = context.astype(jnp.int32)
    negf = neg_context.reshape(-1).astype(jnp.int32)
    vc, ctx, neg = _sc_gather_all(in_embed_w, out_embed_w, center, context,
                                  negf)
    return _tc_score(vc, ctx, neg.reshape(BATCH, KNEG, DIM))
